# baseline (speedup 1.0000x reference)
import functools

import jax
import jax.numpy as jnp
import numpy as np
from jax import lax
from jax.experimental import pallas as pl
from jax.experimental.pallas import tpu as pltpu
from jax.experimental.pallas import tpu_sc as plsc

N = 320000
D = 128
S = 10000
SP = 10112

NC = 2
NS = 16
HC = D // NC
ROWS_PER_T = N // NS
K = 400
NCHUNK = ROWS_PER_T // K
S_PER_TILE = SP // NS
S_LAST = S - (NS - 1) * S_PER_TILE

NB = 2
NGRP = NCHUNK // NB


def _sc_body(x_hbm, ids_hbm, zeros_hbm, out_hbm, acc,
             xbufs, ibufs, semls, semss):
    c = lax.axis_index("c")
    s = lax.axis_index("s")
    row0 = s * ROWS_PER_T
    seg0 = s * S_PER_TILE
    col0 = c * HC

    def start_load(i, b):
        base = row0 + i * K
        pltpu.async_copy(x_hbm.at[pl.ds(base, K), pl.ds(col0, HC)],
                         xbufs[b], semls[b])
        pltpu.async_copy(ids_hbm.at[pl.ds(base, K)], ibufs[b], semls[b])

    def wait_load(b):
        pltpu.make_async_copy(x_hbm.at[pl.ds(0, K), pl.ds(0, HC)],
                              xbufs[b], semls[b]).wait()
        pltpu.make_async_copy(ids_hbm.at[pl.ds(0, K)], ibufs[b],
                              semls[b]).wait()

    def start_scatter(b):
        pltpu.async_copy(xbufs[b], acc.at[ibufs[b]], semss[b], add=True)

    def wait_scatter(b):
        pltpu.make_async_copy(xbufs[b], acc.at[ibufs[b]], semss[b]).wait()

    for b in range(NB):
        start_load(b, b)
    pltpu.sync_copy(zeros_hbm.at[pl.ds(seg0, S_PER_TILE)],
                    acc.at[pl.ds(seg0, S_PER_TILE)])
    plsc.subcore_barrier()

    def body(g, carry):
        for b in range(NB):
            wait_load(b)
            start_scatter(b)
        for b in range(NB):
            wait_scatter(b)
            start_load((g + 1) * NB + b, b)
        return carry

    lax.fori_loop(0, NGRP - 1, body, 0)
    for b in range(NB):
        wait_load(b)
        start_scatter(b)
    for b in range(NB):
        wait_scatter(b)
    plsc.subcore_barrier()

    @pl.when(s < NS - 1)
    def _():
        pltpu.sync_copy(acc.at[pl.ds(seg0, S_PER_TILE)],
                        out_hbm.at[pl.ds(seg0, S_PER_TILE), pl.ds(col0, HC)])

    @pl.when(s == NS - 1)
    def _():
        pltpu.sync_copy(acc.at[pl.ds(seg0, S_LAST)],
                        out_hbm.at[pl.ds(seg0, S_LAST), pl.ds(col0, HC)])


_sc_pool = functools.partial(
    pl.kernel,
    mesh=plsc.VectorSubcoreMesh(core_axis_name="c", subcore_axis_name="s"),
    out_type=jax.ShapeDtypeStruct((S, D), jnp.float32),
    compiler_params=pltpu.CompilerParams(use_tc_tiling_on_sc=False),
    scratch_types=[
        pltpu.VMEM_SHARED((SP, HC), jnp.float32),
        [pltpu.VMEM((K, HC), jnp.float32)] * NB,
        [pltpu.VMEM((K,), jnp.int32)] * NB,
        [pltpu.SemaphoreType.DMA] * NB,
        [pltpu.SemaphoreType.DMA] * NB,
    ],
)(_sc_body)


_ZEROS = np.zeros((SP, HC), np.float32)


def kernel(x, edge_list):
    ids = edge_list.astype(jnp.int32)
    return _sc_pool(x, ids, _ZEROS)


# speedup vs baseline: 1.2142x; 1.2142x over previous
import functools

import jax
import jax.numpy as jnp
import numpy as np
from jax import lax
from jax.experimental import pallas as pl
from jax.experimental.pallas import tpu as pltpu
from jax.experimental.pallas import tpu_sc as plsc

N = 320000
D = 128
S = 10000
SP = 10112

NC = 2
NS = 16
HC = D // NC
ROWS_PER_T = N // NS
K = 200
NCHUNK = ROWS_PER_T // K
S_PER_TILE = SP // NS
S_LAST = S - (NS - 1) * S_PER_TILE

NB = 5
NGRP = NCHUNK // NB


def _sc_body(x_hbm, ids_hbm, zeros_hbm, out_hbm, acc,
             xbufs, ibufs, semls, semss):
    c = lax.axis_index("c")
    s = lax.axis_index("s")
    row0 = s * ROWS_PER_T
    seg0 = s * S_PER_TILE
    col0 = c * HC

    def start_load(i, b):
        base = row0 + i * K
        pltpu.async_copy(x_hbm.at[pl.ds(base, K), pl.ds(col0, HC)],
                         xbufs[b], semls[b])
        pltpu.async_copy(ids_hbm.at[pl.ds(base, K)], ibufs[b], semls[b])

    def wait_load(b):
        pltpu.make_async_copy(x_hbm.at[pl.ds(0, K), pl.ds(0, HC)],
                              xbufs[b], semls[b]).wait()
        pltpu.make_async_copy(ids_hbm.at[pl.ds(0, K)], ibufs[b],
                              semls[b]).wait()

    def start_scatter(b):
        pltpu.async_copy(xbufs[b], acc.at[ibufs[b]], semss[b], add=True)

    def wait_scatter(b):
        pltpu.make_async_copy(xbufs[b], acc.at[ibufs[b]], semss[b]).wait()

    for b in range(NB):
        start_load(b, b)
    pltpu.sync_copy(zeros_hbm.at[pl.ds(seg0, S_PER_TILE)],
                    acc.at[pl.ds(seg0, S_PER_TILE)])
    plsc.subcore_barrier()

    def body(g, carry):
        for b in range(NB):
            wait_load(b)
            start_scatter(b)
        for b in range(NB):
            wait_scatter(b)
            start_load((g + 1) * NB + b, b)
        return carry

    lax.fori_loop(0, NGRP - 1, body, 0)
    for b in range(NB):
        wait_load(b)
        start_scatter(b)
    for b in range(NB):
        wait_scatter(b)
    plsc.subcore_barrier()

    @pl.when(s < NS - 1)
    def _():
        pltpu.sync_copy(acc.at[pl.ds(seg0, S_PER_TILE)],
                        out_hbm.at[pl.ds(seg0, S_PER_TILE), pl.ds(col0, HC)])

    @pl.when(s == NS - 1)
    def _():
        pltpu.sync_copy(acc.at[pl.ds(seg0, S_LAST)],
                        out_hbm.at[pl.ds(seg0, S_LAST), pl.ds(col0, HC)])


_sc_pool = functools.partial(
    pl.kernel,
    mesh=plsc.VectorSubcoreMesh(core_axis_name="c", subcore_axis_name="s"),
    out_type=jax.ShapeDtypeStruct((S, D), jnp.float32),
    compiler_params=pltpu.CompilerParams(use_tc_tiling_on_sc=False),
    scratch_types=[
        pltpu.VMEM_SHARED((SP, HC), jnp.float32),
        [pltpu.VMEM((K, HC), jnp.float32)] * NB,
        [pltpu.VMEM((K,), jnp.int32)] * NB,
        [pltpu.SemaphoreType.DMA] * NB,
        [pltpu.SemaphoreType.DMA] * NB,
    ],
)(_sc_body)


_ZEROS = np.zeros((SP, HC), np.float32)


def kernel(x, edge_list):
    ids = edge_list.astype(jnp.int32)
    return _sc_pool(x, ids, _ZEROS)
